# Initial kernel scaffold; baseline (speedup 1.0000x reference)
import jax
import jax.numpy as jnp
from jax.experimental import pallas as pl


def kernel(q_coord, coord, feat):
    raise NotImplementedError("write your pallas kernel here")


# speedup vs baseline: 4.0577x; 4.0577x over previous
import functools

import jax
import jax.numpy as jnp
from jax import lax
from jax.experimental import pallas as pl
from jax.experimental.pallas import tpu as pltpu
from jax.experimental.pallas import tpu_sc as plsc

M = 4096
N = 16384
C = 128
NSAMPLE = 16
DILATION = 2
K = NSAMPLE * DILATION

QBLK = 256

NC = 2
NS = 16
NW = NC * NS
Q_PER_W = M // NW
GCHUNK = 8
N_CHUNKS = Q_PER_W // GCHUNK


def _topk_body(q_ref, ct_ref, idx_ref, d2_ref):
    q0 = q_ref[:, 0:1]
    q1 = q_ref[:, 1:2]
    q2 = q_ref[:, 2:3]
    c0 = ct_ref[0:1, :]
    c1 = ct_ref[1:2, :]
    c2 = ct_ref[2:3, :]
    sumq = (q0 * q0 + q1 * q1) + q2 * q2
    sumc = (c0 * c0 + c1 * c1) + c2 * c2
    bf = jnp.bfloat16
    f32 = jnp.float32
    q0b = q0.astype(bf).astype(f32)
    q1b = q1.astype(bf).astype(f32)
    q2b = q2.astype(bf).astype(f32)
    c0b = c0.astype(bf).astype(f32)
    c1b = c1.astype(bf).astype(f32)
    c2b = c2.astype(bf).astype(f32)
    dot = (q0b * c0b + q1b * c1b) + q2b * c2b
    d2_ref[...] = (sumq - 2.0 * dot) + sumc

    iota = lax.broadcasted_iota(jnp.int32, (QBLK, N), 1)
    big = jnp.int32(2**30)
    for k in range(K):
        d2v = d2_ref[...]
        minv = jnp.min(d2v, axis=1, keepdims=True)
        cand = jnp.where(d2v == minv, iota, big)
        mi = jnp.min(cand, axis=1, keepdims=True)
        if k % 2 == 0:
            idx_ref[:, (k // 2):(k // 2 + 1)] = mi
        d2_ref[...] = jnp.where(iota == mi, jnp.float32(jnp.inf), d2v)


def _topk_indices(q_coord, coord_t):
    return pl.pallas_call(
        _topk_body,
        grid=(M // QBLK,),
        in_specs=[
            pl.BlockSpec((QBLK, 3), lambda i: (i, 0)),
            pl.BlockSpec((3, N), lambda i: (0, 0)),
        ],
        out_specs=pl.BlockSpec((QBLK, NSAMPLE), lambda i: (i, 0)),
        out_shape=jax.ShapeDtypeStruct((M, NSAMPLE), jnp.int32),
        scratch_shapes=[pltpu.VMEM((QBLK, N), jnp.float32)],
        compiler_params=pltpu.CompilerParams(
            dimension_semantics=("arbitrary",),
        ),
    )(q_coord, coord_t)


def _gather_maxpool(feat, idx_flat):
    mesh = plsc.VectorSubcoreMesh(core_axis_name="core",
                                  subcore_axis_name="subcore")

    @functools.partial(
        pl.kernel,
        out_type=jax.ShapeDtypeStruct((M, C), jnp.float32),
        mesh=mesh,
        scratch_types=[
            pltpu.VMEM((GCHUNK * NSAMPLE,), jnp.int32),
            pltpu.VMEM((GCHUNK * NSAMPLE, C), jnp.float32),
            pltpu.VMEM((GCHUNK, C), jnp.float32),
            pltpu.SemaphoreType.DMA,
        ],
    )
    def kb(feat_hbm, idx_hbm, out_hbm, idx_v, rows_v, out_v, sem):
        wid = lax.axis_index("subcore") * NC + lax.axis_index("core")

        @pl.loop(0, N_CHUNKS)
        def _(ch):
            ibase = wid * (Q_PER_W * NSAMPLE) + ch * (GCHUNK * NSAMPLE)
            pltpu.sync_copy(idx_hbm.at[pl.ds(ibase, GCHUNK * NSAMPLE)], idx_v)
            pltpu.async_copy(feat_hbm.at[idx_v], rows_v, sem).wait()

            @pl.loop(0, GCHUNK)
            def _(g):
                for c in range(C // 16):
                    sl = pl.ds(c * 16, 16)
                    acc = rows_v[g * NSAMPLE, sl]
                    for j in range(1, NSAMPLE):
                        acc = jnp.maximum(acc, rows_v[g * NSAMPLE + j, sl])
                    out_v[g, sl] = acc

            qbase = wid * Q_PER_W + ch * GCHUNK
            pltpu.sync_copy(out_v, out_hbm.at[pl.ds(qbase, GCHUNK)])

    return kb(feat, idx_flat)


def kernel(q_coord, coord, feat):
    coord_t = coord.T
    idx = _topk_indices(q_coord, coord_t)
    return _gather_maxpool(feat, idx.reshape(-1))


# speedup vs baseline: 4.0600x; 1.0006x over previous
import functools

import jax
import jax.numpy as jnp
from jax import lax
from jax.experimental import pallas as pl
from jax.experimental.pallas import tpu as pltpu
from jax.experimental.pallas import tpu_sc as plsc

M = 4096
N = 16384
C = 128
NSAMPLE = 16
DILATION = 2
K = NSAMPLE * DILATION

QBLK = 256

NC = 2
NS = 16
NW = NC * NS
Q_PER_W = M // NW
GCHUNK = 8
N_CHUNKS = Q_PER_W // GCHUNK


def _topk_body(q_ref, ct_ref, idx_ref, d2_ref):
    q0 = q_ref[:, 0:1]
    q1 = q_ref[:, 1:2]
    q2 = q_ref[:, 2:3]
    c0 = ct_ref[0:1, :]
    c1 = ct_ref[1:2, :]
    c2 = ct_ref[2:3, :]
    sumq = (q0 * q0 + q1 * q1) + q2 * q2
    sumc = (c0 * c0 + c1 * c1) + c2 * c2
    bf = jnp.bfloat16
    f32 = jnp.float32
    q0b = q0.astype(bf).astype(f32)
    q1b = q1.astype(bf).astype(f32)
    q2b = q2.astype(bf).astype(f32)
    c0b = c0.astype(bf).astype(f32)
    c1b = c1.astype(bf).astype(f32)
    c2b = c2.astype(bf).astype(f32)
    dot = (q0b * c0b + q1b * c1b) + q2b * c2b
    d2_ref[...] = (sumq - 2.0 * dot) + sumc

    iota = lax.broadcasted_iota(jnp.int32, (QBLK, N), 1)
    big = jnp.int32(2**30)
    for k in range(K):
        d2v = d2_ref[...]
        minv = jnp.min(d2v, axis=1, keepdims=True)
        cand = jnp.where(d2v == minv, iota, big)
        mi = jnp.min(cand, axis=1, keepdims=True)
        if k % 2 == 0:
            idx_ref[:, (k // 2):(k // 2 + 1)] = mi
        d2_ref[...] = jnp.where(iota == mi, jnp.float32(jnp.inf), d2v)


def _topk_indices(q_coord, coord_t):
    return pl.pallas_call(
        _topk_body,
        grid=(M // QBLK,),
        in_specs=[
            pl.BlockSpec((QBLK, 3), lambda i: (i, 0)),
            pl.BlockSpec((3, N), lambda i: (0, 0)),
        ],
        out_specs=pl.BlockSpec((QBLK, NSAMPLE), lambda i: (i, 0)),
        out_shape=jax.ShapeDtypeStruct((M, NSAMPLE), jnp.int32),
        scratch_shapes=[pltpu.VMEM((QBLK, N), jnp.float32)],
        compiler_params=pltpu.CompilerParams(
            dimension_semantics=("parallel",),
        ),
    )(q_coord, coord_t)


def _gather_maxpool(feat, idx_flat):
    mesh = plsc.VectorSubcoreMesh(core_axis_name="core",
                                  subcore_axis_name="subcore")

    @functools.partial(
        pl.kernel,
        out_type=jax.ShapeDtypeStruct((M, C), jnp.float32),
        mesh=mesh,
        scratch_types=[
            pltpu.VMEM((GCHUNK * NSAMPLE,), jnp.int32),
            pltpu.VMEM((GCHUNK * NSAMPLE, C), jnp.float32),
            pltpu.VMEM((GCHUNK, C), jnp.float32),
            pltpu.SemaphoreType.DMA,
        ],
    )
    def kb(feat_hbm, idx_hbm, out_hbm, idx_v, rows_v, out_v, sem):
        wid = lax.axis_index("subcore") * NC + lax.axis_index("core")

        @pl.loop(0, N_CHUNKS)
        def _(ch):
            ibase = wid * (Q_PER_W * NSAMPLE) + ch * (GCHUNK * NSAMPLE)
            pltpu.sync_copy(idx_hbm.at[pl.ds(ibase, GCHUNK * NSAMPLE)], idx_v)
            pltpu.async_copy(feat_hbm.at[idx_v], rows_v, sem).wait()

            @pl.loop(0, GCHUNK)
            def _(g):
                for c in range(C // 16):
                    sl = pl.ds(c * 16, 16)
                    acc = rows_v[g * NSAMPLE, sl]
                    for j in range(1, NSAMPLE):
                        acc = jnp.maximum(acc, rows_v[g * NSAMPLE + j, sl])
                    out_v[g, sl] = acc

            qbase = wid * Q_PER_W + ch * GCHUNK
            pltpu.sync_copy(out_v, out_hbm.at[pl.ds(qbase, GCHUNK)])

    return kb(feat, idx_flat)


def kernel(q_coord, coord, feat):
    coord_t = coord.T
    idx = _topk_indices(q_coord, coord_t)
    return _gather_maxpool(feat, idx.reshape(-1))


# speedup vs baseline: 6.7490x; 1.6623x over previous
import functools

import jax
import jax.numpy as jnp
from jax import lax
from jax.experimental import pallas as pl
from jax.experimental.pallas import tpu as pltpu
from jax.experimental.pallas import tpu_sc as plsc

M = 4096
N = 16384
C = 128
NSAMPLE = 16
DILATION = 2
K = NSAMPLE * DILATION

QBLK = 128

NC = 2
NS = 16
NW = NC * NS
Q_PER_W = M // NW
GCHUNK = 8
N_CHUNKS = Q_PER_W // GCHUNK


VDIM = 128
LDIM = 128
SLOTS = 8
import numpy as np
BIGI = np.int32(2**30)
INF = np.float32(np.inf)


def _compute_d2(q_ref, ct_ref):
    q0 = q_ref[:, 0:1].reshape(QBLK, 1, 1)
    q1 = q_ref[:, 1:2].reshape(QBLK, 1, 1)
    q2 = q_ref[:, 2:3].reshape(QBLK, 1, 1)
    c0 = ct_ref[0:1]
    c1 = ct_ref[1:2]
    c2 = ct_ref[2:3]
    sumq = (q0 * q0 + q1 * q1) + q2 * q2
    sumc = (c0 * c0 + c1 * c1) + c2 * c2
    bf = jnp.bfloat16
    f32 = jnp.float32
    q0b = q0.astype(bf).astype(f32)
    q1b = q1.astype(bf).astype(f32)
    q2b = q2.astype(bf).astype(f32)
    c0b = c0.astype(bf).astype(f32)
    c1b = c1.astype(bf).astype(f32)
    c2b = c2.astype(bf).astype(f32)
    dot = (q0b * c0b + q1b * c1b) + q2b * c2b
    return (sumq - 2.0 * dot) + sumc


def _topk_body(q_ref, ct_ref, idx_ref, d2_ref, cval_ref, cn_ref):
    d2_ref[...] = _compute_d2(q_ref, ct_ref)

    iota_v = lax.broadcasted_iota(jnp.int32, (QBLK, VDIM, LDIM), 1)
    iota_l2 = lax.broadcasted_iota(jnp.int32, (QBLK, LDIM), 1)
    for s in range(SLOTS):
        dv = d2_ref[...]
        pmin = jnp.min(dv, axis=1)
        candv = jnp.where(dv == pmin[:, None, :], iota_v, BIGI)
        vmin = jnp.min(candv, axis=1)
        d2_ref[...] = jnp.where(iota_v == vmin[:, None, :], INF, dv)
        n = vmin * LDIM + iota_l2
        cval_ref[s * LDIM:(s + 1) * LDIM, :] = pmin.T
        cn_ref[s * LDIM:(s + 1) * LDIM, :] = n.T

    pops = []
    for k in range(K):
        vals = cval_ref[...]
        minv = jnp.min(vals, axis=0, keepdims=True)
        candn = jnp.where(vals == minv, cn_ref[...], BIGI)
        mi = jnp.min(candn, axis=0, keepdims=True)
        if k % DILATION == 0:
            pops.append(mi)
        cval_ref[...] = jnp.where(cn_ref[...] == mi, INF, vals)
    idx_ref[...] = jnp.concatenate(pops, axis=0).T

    popped = (cval_ref[...] == INF).astype(jnp.int32)
    cnt = popped[0:LDIM, :]
    for s in range(1, SLOTS):
        cnt = cnt + popped[s * LDIM:(s + 1) * LDIM, :]
    flag = jnp.max(jnp.where(cnt >= SLOTS, 1, 0))

    @pl.when(flag > 0)
    def _fallback():
        d2_ref[...] = _compute_d2(q_ref, ct_ref)
        for k in range(K):
            dv = d2_ref[...]
            m = jnp.min(jnp.min(dv, axis=1), axis=1, keepdims=True)
            n3 = iota_v * LDIM + lax.broadcasted_iota(
                jnp.int32, (QBLK, VDIM, LDIM), 2)
            cand = jnp.where(dv == m[:, :, None], n3, BIGI)
            mi = jnp.min(jnp.min(cand, axis=1), axis=1, keepdims=True)
            if k % DILATION == 0:
                idx_ref[:, (k // DILATION):(k // DILATION + 1)] = mi
            d2_ref[...] = jnp.where(n3 == mi[:, :, None], INF, dv)


def _topk_indices(q_coord, coord_t3):
    return pl.pallas_call(
        _topk_body,
        grid=(M // QBLK,),
        in_specs=[
            pl.BlockSpec((QBLK, 3), lambda i: (i, 0)),
            pl.BlockSpec((3, VDIM, LDIM), lambda i: (0, 0, 0)),
        ],
        out_specs=pl.BlockSpec((QBLK, NSAMPLE), lambda i: (i, 0)),
        out_shape=jax.ShapeDtypeStruct((M, NSAMPLE), jnp.int32),
        scratch_shapes=[
            pltpu.VMEM((QBLK, VDIM, LDIM), jnp.float32),
            pltpu.VMEM((SLOTS * LDIM, QBLK), jnp.float32),
            pltpu.VMEM((SLOTS * LDIM, QBLK), jnp.int32),
        ],
        compiler_params=pltpu.CompilerParams(
            dimension_semantics=("parallel",),
        ),
    )(q_coord, coord_t3)


def _gather_maxpool(feat, idx_flat):
    mesh = plsc.VectorSubcoreMesh(core_axis_name="core",
                                  subcore_axis_name="subcore")

    @functools.partial(
        pl.kernel,
        out_type=jax.ShapeDtypeStruct((M, C), jnp.float32),
        mesh=mesh,
        scratch_types=[
            pltpu.VMEM((GCHUNK * NSAMPLE,), jnp.int32),
            pltpu.VMEM((GCHUNK * NSAMPLE, C), jnp.float32),
            pltpu.VMEM((GCHUNK, C), jnp.float32),
            pltpu.SemaphoreType.DMA,
        ],
    )
    def kb(feat_hbm, idx_hbm, out_hbm, idx_v, rows_v, out_v, sem):
        wid = lax.axis_index("subcore") * NC + lax.axis_index("core")

        @pl.loop(0, N_CHUNKS)
        def _(ch):
            ibase = wid * (Q_PER_W * NSAMPLE) + ch * (GCHUNK * NSAMPLE)
            pltpu.sync_copy(idx_hbm.at[pl.ds(ibase, GCHUNK * NSAMPLE)], idx_v)
            pltpu.async_copy(feat_hbm.at[idx_v], rows_v, sem).wait()

            @pl.loop(0, GCHUNK)
            def _(g):
                for c in range(C // 16):
                    sl = pl.ds(c * 16, 16)
                    acc = rows_v[g * NSAMPLE, sl]
                    for j in range(1, NSAMPLE):
                        acc = jnp.maximum(acc, rows_v[g * NSAMPLE + j, sl])
                    out_v[g, sl] = acc

            qbase = wid * Q_PER_W + ch * GCHUNK
            pltpu.sync_copy(out_v, out_hbm.at[pl.ds(qbase, GCHUNK)])

    return kb(feat, idx_flat)


def kernel(q_coord, coord, feat):
    coord_t3 = coord.T.reshape(3, VDIM, LDIM)
    idx = _topk_indices(q_coord, coord_t3)
    return _gather_maxpool(feat, idx.reshape(-1))


# speedup vs baseline: 11.5382x; 1.7096x over previous
import functools

import jax
import jax.numpy as jnp
from jax import lax
from jax.experimental import pallas as pl
from jax.experimental.pallas import tpu as pltpu
from jax.experimental.pallas import tpu_sc as plsc

M = 4096
N = 16384
C = 128
NSAMPLE = 16
DILATION = 2
K = NSAMPLE * DILATION

QBLK = 128

NC = 2
NS = 16
NW = NC * NS
Q_PER_W = M // NW
GCHUNK = 8
N_CHUNKS = Q_PER_W // GCHUNK


VDIM = 128
LDIM = 128
SLOTS = 6
import numpy as np
BIGI = np.int32(2**30)
INF = np.float32(np.inf)


def _compute_d2(q_ref, ct_ref):
    q0 = q_ref[:, 0:1].reshape(QBLK, 1, 1)
    q1 = q_ref[:, 1:2].reshape(QBLK, 1, 1)
    q2 = q_ref[:, 2:3].reshape(QBLK, 1, 1)
    c0 = ct_ref[0:1]
    c1 = ct_ref[1:2]
    c2 = ct_ref[2:3]
    sumq = (q0 * q0 + q1 * q1) + q2 * q2
    sumc = (c0 * c0 + c1 * c1) + c2 * c2
    bf = jnp.bfloat16
    f32 = jnp.float32
    q0b = q0.astype(bf).astype(f32)
    q1b = q1.astype(bf).astype(f32)
    q2b = q2.astype(bf).astype(f32)
    c0b = c0.astype(bf).astype(f32)
    c1b = c1.astype(bf).astype(f32)
    c2b = c2.astype(bf).astype(f32)
    dot = (q0b * c0b + q1b * c1b) + q2b * c2b
    return (sumq - 2.0 * dot) + sumc


def _topk_body(q_ref, ct_ref, idx_ref, d2_ref, cval_ref, cn_ref):
    d2_ref[...] = _compute_d2(q_ref, ct_ref)

    iota_v = lax.broadcasted_iota(jnp.int32, (QBLK, VDIM, LDIM), 1)
    iota_l2 = lax.broadcasted_iota(jnp.int32, (QBLK, LDIM), 1)
    for s in range(SLOTS):
        dv = d2_ref[...]
        pmin = jnp.min(dv, axis=1)
        candv = jnp.where(dv == pmin[:, None, :], iota_v, BIGI)
        vmin = jnp.min(candv, axis=1)
        d2_ref[...] = jnp.where(iota_v == vmin[:, None, :], INF, dv)
        n = vmin * LDIM + iota_l2
        cval_ref[s * LDIM:(s + 1) * LDIM, :] = pmin.T
        cn_ref[s * LDIM:(s + 1) * LDIM, :] = n.T

    pops = []
    for k in range(K):
        vals = cval_ref[...]
        minv = jnp.min(vals, axis=0, keepdims=True)
        candn = jnp.where(vals == minv, cn_ref[...], BIGI)
        mi = jnp.min(candn, axis=0, keepdims=True)
        if k % DILATION == 0:
            pops.append(mi)
        cval_ref[...] = jnp.where(cn_ref[...] == mi, INF, vals)
    idx_ref[...] = jnp.concatenate(pops, axis=0).T

    popped = (cval_ref[...] == INF).astype(jnp.int32)
    cnt = popped[0:LDIM, :]
    for s in range(1, SLOTS):
        cnt = cnt + popped[s * LDIM:(s + 1) * LDIM, :]
    flag = jnp.max(jnp.where(cnt >= SLOTS, 1, 0))

    @pl.when(flag > 0)
    def _fallback():
        d2_ref[...] = _compute_d2(q_ref, ct_ref)
        for k in range(K):
            dv = d2_ref[...]
            m = jnp.min(jnp.min(dv, axis=1), axis=1, keepdims=True)
            n3 = iota_v * LDIM + lax.broadcasted_iota(
                jnp.int32, (QBLK, VDIM, LDIM), 2)
            cand = jnp.where(dv == m[:, :, None], n3, BIGI)
            mi = jnp.min(jnp.min(cand, axis=1), axis=1, keepdims=True)
            if k % DILATION == 0:
                idx_ref[:, (k // DILATION):(k // DILATION + 1)] = mi
            d2_ref[...] = jnp.where(n3 == mi[:, :, None], INF, dv)


def _topk_indices(q_coord, coord_t3):
    return pl.pallas_call(
        _topk_body,
        grid=(M // QBLK,),
        in_specs=[
            pl.BlockSpec((QBLK, 3), lambda i: (i, 0)),
            pl.BlockSpec((3, VDIM, LDIM), lambda i: (0, 0, 0)),
        ],
        out_specs=pl.BlockSpec((QBLK, NSAMPLE), lambda i: (i, 0)),
        out_shape=jax.ShapeDtypeStruct((M, NSAMPLE), jnp.int32),
        scratch_shapes=[
            pltpu.VMEM((QBLK, VDIM, LDIM), jnp.float32),
            pltpu.VMEM((SLOTS * LDIM, QBLK), jnp.float32),
            pltpu.VMEM((SLOTS * LDIM, QBLK), jnp.int32),
        ],
        compiler_params=pltpu.CompilerParams(
            dimension_semantics=("parallel",),
        ),
    )(q_coord, coord_t3)


def _gather_maxpool(feat, idx_flat):
    mesh = plsc.VectorSubcoreMesh(core_axis_name="core",
                                  subcore_axis_name="subcore")

    @functools.partial(
        pl.kernel,
        out_type=jax.ShapeDtypeStruct((M, C), jnp.float32),
        mesh=mesh,
        scratch_types=[
            pltpu.VMEM((GCHUNK * NSAMPLE,), jnp.int32),
            pltpu.VMEM((GCHUNK * NSAMPLE, C), jnp.float32),
            pltpu.VMEM((GCHUNK, C), jnp.float32),
            pltpu.SemaphoreType.DMA,
        ],
    )
    def kb(feat_hbm, idx_hbm, out_hbm, idx_v, rows_v, out_v, sem):
        wid = lax.axis_index("subcore") * NC + lax.axis_index("core")

        @pl.loop(0, N_CHUNKS)
        def _(ch):
            ibase = wid * (Q_PER_W * NSAMPLE) + ch * (GCHUNK * NSAMPLE)
            pltpu.sync_copy(idx_hbm.at[pl.ds(ibase, GCHUNK * NSAMPLE)], idx_v)
            pltpu.async_copy(feat_hbm.at[idx_v], rows_v, sem).wait()

            @pl.loop(0, GCHUNK)
            def _(g):
                for c in range(C // 16):
                    sl = pl.ds(c * 16, 16)
                    acc = rows_v[g * NSAMPLE, sl]
                    for j in range(1, NSAMPLE):
                        acc = jnp.maximum(acc, rows_v[g * NSAMPLE + j, sl])
                    out_v[g, sl] = acc

            qbase = wid * Q_PER_W + ch * GCHUNK
            pltpu.sync_copy(out_v, out_hbm.at[pl.ds(qbase, GCHUNK)])

    return kb(feat, idx_flat)


def kernel(q_coord, coord, feat):
    coord_t3 = coord.T.reshape(3, VDIM, LDIM)
    idx = _topk_indices(q_coord, coord_t3)
    return _gather_maxpool(feat, idx.reshape(-1))


# speedup vs baseline: 13.3893x; 1.1604x over previous
import functools

import jax
import jax.numpy as jnp
from jax import lax
from jax.experimental import pallas as pl
from jax.experimental.pallas import tpu as pltpu
from jax.experimental.pallas import tpu_sc as plsc

M = 4096
N = 16384
C = 128
NSAMPLE = 16
DILATION = 2
K = NSAMPLE * DILATION

QBLK = 128

NC = 2
NS = 16
NW = NC * NS
Q_PER_W = M // NW
GCHUNK = 8
N_CHUNKS = Q_PER_W // GCHUNK


VDIM = 128
LDIM = 128
SLOTS = 5
import numpy as np
BIGI = np.int32(2**30)
INF = np.float32(np.inf)


def _compute_d2(q_ref, ct_ref):
    q0 = q_ref[:, 0:1].reshape(QBLK, 1, 1)
    q1 = q_ref[:, 1:2].reshape(QBLK, 1, 1)
    q2 = q_ref[:, 2:3].reshape(QBLK, 1, 1)
    c0 = ct_ref[0:1]
    c1 = ct_ref[1:2]
    c2 = ct_ref[2:3]
    sumq = (q0 * q0 + q1 * q1) + q2 * q2
    sumc = (c0 * c0 + c1 * c1) + c2 * c2
    bf = jnp.bfloat16
    f32 = jnp.float32
    q0b = q0.astype(bf).astype(f32)
    q1b = q1.astype(bf).astype(f32)
    q2b = q2.astype(bf).astype(f32)
    c0b = c0.astype(bf).astype(f32)
    c1b = c1.astype(bf).astype(f32)
    c2b = c2.astype(bf).astype(f32)
    dot = (q0b * c0b + q1b * c1b) + q2b * c2b
    return (sumq - 2.0 * dot) + sumc


def _topk_body(q_ref, ct_ref, idx_ref, d2_ref, cval_ref, cn_ref):
    d2_ref[...] = _compute_d2(q_ref, ct_ref)

    iota_v = lax.broadcasted_iota(jnp.int32, (QBLK, VDIM, LDIM), 1)
    iota_l2 = lax.broadcasted_iota(jnp.int32, (QBLK, LDIM), 1)
    for s in range(SLOTS):
        dv = d2_ref[...]
        pmin = jnp.min(dv, axis=1)
        candv = jnp.where(dv == pmin[:, None, :], iota_v, BIGI)
        vmin = jnp.min(candv, axis=1)
        d2_ref[...] = jnp.where(iota_v == vmin[:, None, :], INF, dv)
        n = vmin * LDIM + iota_l2
        cval_ref[s * LDIM:(s + 1) * LDIM, :] = pmin.T
        cn_ref[s * LDIM:(s + 1) * LDIM, :] = n.T

    pops = []
    for k in range(K):
        vals = cval_ref[...]
        minv = jnp.min(vals, axis=0, keepdims=True)
        candn = jnp.where(vals == minv, cn_ref[...], BIGI)
        mi = jnp.min(candn, axis=0, keepdims=True)
        if k % DILATION == 0:
            pops.append(mi)
        cval_ref[...] = jnp.where(cn_ref[...] == mi, INF, vals)
    idx_ref[...] = jnp.concatenate(pops, axis=0).T

    popped = (cval_ref[...] == INF).astype(jnp.int32)
    cnt = popped[0:LDIM, :]
    for s in range(1, SLOTS):
        cnt = cnt + popped[s * LDIM:(s + 1) * LDIM, :]
    flag = jnp.max(jnp.where(cnt >= SLOTS, 1, 0))

    @pl.when(flag > 0)
    def _fallback():
        d2_ref[...] = _compute_d2(q_ref, ct_ref)
        for k in range(K):
            dv = d2_ref[...]
            m = jnp.min(jnp.min(dv, axis=1), axis=1, keepdims=True)
            n3 = iota_v * LDIM + lax.broadcasted_iota(
                jnp.int32, (QBLK, VDIM, LDIM), 2)
            cand = jnp.where(dv == m[:, :, None], n3, BIGI)
            mi = jnp.min(jnp.min(cand, axis=1), axis=1, keepdims=True)
            if k % DILATION == 0:
                idx_ref[:, (k // DILATION):(k // DILATION + 1)] = mi
            d2_ref[...] = jnp.where(n3 == mi[:, :, None], INF, dv)


def _topk_indices(q_coord, coord_t3):
    return pl.pallas_call(
        _topk_body,
        grid=(M // QBLK,),
        in_specs=[
            pl.BlockSpec((QBLK, 3), lambda i: (i, 0)),
            pl.BlockSpec((3, VDIM, LDIM), lambda i: (0, 0, 0)),
        ],
        out_specs=pl.BlockSpec((QBLK, NSAMPLE), lambda i: (i, 0)),
        out_shape=jax.ShapeDtypeStruct((M, NSAMPLE), jnp.int32),
        scratch_shapes=[
            pltpu.VMEM((QBLK, VDIM, LDIM), jnp.float32),
            pltpu.VMEM((SLOTS * LDIM, QBLK), jnp.float32),
            pltpu.VMEM((SLOTS * LDIM, QBLK), jnp.int32),
        ],
        compiler_params=pltpu.CompilerParams(
            dimension_semantics=("parallel",),
        ),
    )(q_coord, coord_t3)


def _gather_maxpool(feat, idx_flat):
    mesh = plsc.VectorSubcoreMesh(core_axis_name="core",
                                  subcore_axis_name="subcore")

    @functools.partial(
        pl.kernel,
        out_type=jax.ShapeDtypeStruct((M, C), jnp.float32),
        mesh=mesh,
        scratch_types=[
            pltpu.VMEM((GCHUNK * NSAMPLE,), jnp.int32),
            pltpu.VMEM((GCHUNK * NSAMPLE, C), jnp.float32),
            pltpu.VMEM((GCHUNK, C), jnp.float32),
            pltpu.SemaphoreType.DMA,
        ],
    )
    def kb(feat_hbm, idx_hbm, out_hbm, idx_v, rows_v, out_v, sem):
        wid = lax.axis_index("subcore") * NC + lax.axis_index("core")

        @pl.loop(0, N_CHUNKS)
        def _(ch):
            ibase = wid * (Q_PER_W * NSAMPLE) + ch * (GCHUNK * NSAMPLE)
            pltpu.sync_copy(idx_hbm.at[pl.ds(ibase, GCHUNK * NSAMPLE)], idx_v)
            pltpu.async_copy(feat_hbm.at[idx_v], rows_v, sem).wait()

            @pl.loop(0, GCHUNK)
            def _(g):
                for c in range(C // 16):
                    sl = pl.ds(c * 16, 16)
                    acc = rows_v[g * NSAMPLE, sl]
                    for j in range(1, NSAMPLE):
                        acc = jnp.maximum(acc, rows_v[g * NSAMPLE + j, sl])
                    out_v[g, sl] = acc

            qbase = wid * Q_PER_W + ch * GCHUNK
            pltpu.sync_copy(out_v, out_hbm.at[pl.ds(qbase, GCHUNK)])

    return kb(feat, idx_flat)


def kernel(q_coord, coord, feat):
    coord_t3 = coord.T.reshape(3, VDIM, LDIM)
    idx = _topk_indices(q_coord, coord_t3)
    return _gather_maxpool(feat, idx.reshape(-1))
